# baseline (speedup 1.0000x reference)
import jax
import jax.numpy as jnp
from jax.experimental import pallas as pl
from jax.experimental.pallas import tpu as pltpu

_EPS = 1e-06
_TEMP = 20.0
_PAD = 8


def _fused_body(feats_ref, labels_ref, q_ref, out_ref, pm_ref):
    i = pl.program_id(0)

    @pl.when(i == 0)
    def _support():
        n, E = feats_ref.shape
        C = labels_ref.shape[0]
        nch = 8
        ch = n // nch
        musum = jnp.zeros((1, E), jnp.float32)
        for k in range(nch):
            musum = musum + jnp.sum(
                feats_ref[pl.ds(k * ch, ch), :], axis=0, keepdims=True)
        mu = musum / n
        ps = jnp.zeros((C, E), jnp.float32)
        counts = jnp.zeros((C,), jnp.float32)
        for k in range(nch):
            y = feats_ref[pl.ds(k * ch, ch), :] - mu
            ss = jnp.sum(y * y, axis=1, keepdims=True)
            yn = y / jnp.maximum(jnp.sqrt(ss), 1e-12)
            w = (labels_ref[:, pl.ds(k * ch, ch)] > 0.5).astype(jnp.float32)
            counts = counts + jnp.sum(w, axis=1)
            ps = ps + jax.lax.dot_general(
                w, yn, (((1,), (0,)), ((), ())))
        protos = ps / jnp.maximum(counts, _EPS)[:, None]
        pn = jnp.sqrt(jnp.sum(protos * protos, axis=1, keepdims=True))
        protos = protos / jnp.maximum(pn, 1e-12)
        mup = jax.lax.dot_general(
            mu, protos, (((1,), (1,)), ((), ())))
        mumu = jnp.sum(mu * mu, axis=1, keepdims=True)
        aux = jnp.concatenate(
            [mup, mumu, jnp.zeros((1, E - C - 1), jnp.float32)], axis=1)
        pm_ref[...] = jnp.concatenate([protos, mu, aux], axis=0)

    @pl.when(i > 0)
    def _query():
        x = q_ref[...]
        pm = pm_ref[...]
        aux = pm[_PAD - 1:_PAD, 0:_PAD]
        mumu = pm[_PAD - 1:_PAD, _PAD - 2:_PAD - 1]
        ss = jnp.sum(x * x, axis=1, keepdims=True)
        dots = jax.lax.dot_general(
            x, pm, (((1,), (1,)), ((), ())))
        xmu = dots[:, _PAD - 2:_PAD - 1]
        d2 = jnp.maximum(ss - 2.0 * xmu + mumu, 0.0)
        scale = _TEMP / jnp.maximum(jnp.sqrt(d2), 1e-12)
        C = out_ref.shape[1]
        out_ref[...] = (dots[:, :C] - aux[:, :C]) * scale


def kernel(support_feats, support_labels, query_feats):
    E = support_feats.shape[-1]
    C = support_labels.shape[-1]
    feats = support_feats.reshape(-1, E)
    labels = support_labels.reshape(-1, C).T
    q = query_feats.reshape(-1, E)
    nq = q.shape[0]
    n = feats.shape[0]

    B = 2048
    nblk = nq // B
    out = pl.pallas_call(
        _fused_body,
        grid=(1 + nblk,),
        in_specs=[
            pl.BlockSpec((n, E), lambda i: (0, 0)),
            pl.BlockSpec((C, n), lambda i: (0, 0)),
            pl.BlockSpec((B, E), lambda i: (jnp.maximum(i - 1, 0), 0)),
        ],
        out_specs=pl.BlockSpec((B, C), lambda i: (jnp.maximum(i - 1, 0), 0)),
        out_shape=jax.ShapeDtypeStruct((nq, C), jnp.float32),
        scratch_shapes=[pltpu.VMEM((_PAD, E), jnp.float32)],
        compiler_params=pltpu.CompilerParams(
            vmem_limit_bytes=63 * 1024 * 1024),
    )(feats, labels, q)
    return out


# speedup vs baseline: 1.0385x; 1.0385x over previous
import jax
import jax.numpy as jnp
from jax.experimental import pallas as pl
from jax.experimental.pallas import tpu as pltpu

_EPS = 1e-06
_TEMP = 20.0
_PAD = 8


def _query_block(x, pm):
    aux = pm[_PAD - 1:_PAD, 0:_PAD]
    mumu = pm[_PAD - 1:_PAD, _PAD - 2:_PAD - 1]
    ss = jnp.sum(x * x, axis=1, keepdims=True)
    dots = jax.lax.dot_general(x, pm, (((1,), (1,)), ((), ())))
    xmu = dots[:, _PAD - 2:_PAD - 1]
    d2 = jnp.maximum(ss - 2.0 * xmu + mumu, 0.0)
    scale = _TEMP / jnp.maximum(jnp.sqrt(d2), 1e-12)
    return dots, aux, scale


def _fused_body(feats_ref, labels_ref, q1_ref, q2_ref, out_ref, pm_ref):
    i = pl.program_id(0)

    @pl.when(i == 0)
    def _support():
        n, E = feats_ref.shape
        C = labels_ref.shape[0]
        nch = 8
        ch = n // nch
        musum = jnp.zeros((1, E), jnp.float32)
        for k in range(nch):
            musum = musum + jnp.sum(
                feats_ref[pl.ds(k * ch, ch), :], axis=0, keepdims=True)
        mu = musum / n
        ps = jnp.zeros((C, E), jnp.float32)
        counts = jnp.zeros((C,), jnp.float32)
        for k in range(nch):
            y = feats_ref[pl.ds(k * ch, ch), :] - mu
            ss = jnp.sum(y * y, axis=1, keepdims=True)
            yn = y / jnp.maximum(jnp.sqrt(ss), 1e-12)
            w = (labels_ref[:, pl.ds(k * ch, ch)] > 0.5).astype(jnp.float32)
            counts = counts + jnp.sum(w, axis=1)
            ps = ps + jax.lax.dot_general(
                w, yn, (((1,), (0,)), ((), ())))
        protos = ps / jnp.maximum(counts, _EPS)[:, None]
        pn = jnp.sqrt(jnp.sum(protos * protos, axis=1, keepdims=True))
        protos = protos / jnp.maximum(pn, 1e-12)
        mup = jax.lax.dot_general(mu, protos, (((1,), (1,)), ((), ())))
        mumu = jnp.sum(mu * mu, axis=1, keepdims=True)
        aux = jnp.concatenate(
            [mup, mumu, jnp.zeros((1, E - C - 1), jnp.float32)], axis=1)
        pm_ref[...] = jnp.concatenate([protos, mu, aux], axis=0)

    @pl.when(i > 0)
    def _query():
        pm = pm_ref[...]
        B, C = out_ref.shape
        B = B // 2
        dots1, aux1, scale1 = _query_block(q1_ref[...], pm)
        out_ref[0:B, :] = (dots1[:, :C] - aux1[:, :C]) * scale1
        dots2, aux2, scale2 = _query_block(q2_ref[...], pm)
        out_ref[B:2 * B, :] = (dots2[:, :C] - aux2[:, :C]) * scale2


def kernel(support_feats, support_labels, query_feats):
    E = support_feats.shape[-1]
    C = support_labels.shape[-1]
    feats = support_feats.reshape(-1, E)
    labels = support_labels.reshape(-1, C).T
    q = query_feats.reshape(-1, E)
    nq = q.shape[0]
    n = feats.shape[0]

    B = 2048
    nblk = nq // (2 * B)
    out = pl.pallas_call(
        _fused_body,
        grid=(1 + nblk,),
        in_specs=[
            pl.BlockSpec((n, E), lambda i: (0, 0)),
            pl.BlockSpec((C, n), lambda i: (0, 0)),
            pl.BlockSpec((B, E), lambda i: (2 * jnp.maximum(i - 1, 0), 0)),
            pl.BlockSpec((B, E), lambda i: (2 * jnp.maximum(i - 1, 0) + 1, 0)),
        ],
        out_specs=pl.BlockSpec((2 * B, C), lambda i: (jnp.maximum(i - 1, 0), 0)),
        out_shape=jax.ShapeDtypeStruct((nq, C), jnp.float32),
        scratch_shapes=[pltpu.VMEM((_PAD, E), jnp.float32)],
        compiler_params=pltpu.CompilerParams(
            vmem_limit_bytes=63 * 1024 * 1024),
    )(feats, labels, q, q)
    return out


# speedup vs baseline: 1.0473x; 1.0084x over previous
import jax
import jax.numpy as jnp
from jax.experimental import pallas as pl
from jax.experimental.pallas import tpu as pltpu

_EPS = 1e-06
_TEMP = 20.0
_PAD = 8


def _fused_body(feats_ref, labels_ref, q_ref, out_ref, pm_ref):
    i = pl.program_id(0)

    @pl.when(i == 0)
    def _support():
        n, E = feats_ref.shape
        C = labels_ref.shape[0]
        nch = 8
        ch = n // nch
        musum = jnp.zeros((1, E), jnp.float32)
        for k in range(nch):
            musum = musum + jnp.sum(
                feats_ref[pl.ds(k * ch, ch), :], axis=0, keepdims=True)
        mu = musum / n
        ps = jnp.zeros((C, E), jnp.float32)
        counts = jnp.zeros((C,), jnp.float32)
        for k in range(nch):
            y = feats_ref[pl.ds(k * ch, ch), :] - mu
            ss = jnp.sum(y * y, axis=1, keepdims=True)
            yn = y / jnp.maximum(jnp.sqrt(ss), 1e-12)
            w = (labels_ref[:, pl.ds(k * ch, ch)] > 0.5).astype(jnp.float32)
            counts = counts + jnp.sum(w, axis=1)
            ps = ps + jax.lax.dot_general(
                w, yn, (((1,), (0,)), ((), ())))
        protos = ps / jnp.maximum(counts, _EPS)[:, None]
        pn = jnp.sqrt(jnp.sum(protos * protos, axis=1, keepdims=True))
        protos = protos / jnp.maximum(pn, 1e-12)
        mup = jax.lax.dot_general(
            mu, protos, (((1,), (1,)), ((), ())))
        mumu = jnp.sum(mu * mu, axis=1, keepdims=True)
        aux = jnp.concatenate(
            [mup, mumu, jnp.zeros((1, E - C - 1), jnp.float32)], axis=1)
        pm_ref[...] = jnp.concatenate([protos, mu, aux], axis=0)

    @pl.when(i > 0)
    def _query():
        x = q_ref[...]
        pm = pm_ref[...]
        aux = pm[_PAD - 1:_PAD, 0:_PAD]
        mumu = pm[_PAD - 1:_PAD, _PAD - 2:_PAD - 1]
        ss = jnp.sum(x * x, axis=1, keepdims=True)
        dots = jax.lax.dot_general(
            x, pm, (((1,), (1,)), ((), ())))
        xmu = dots[:, _PAD - 2:_PAD - 1]
        d2 = jnp.maximum(ss - 2.0 * xmu + mumu, 0.0)
        scale = _TEMP / jnp.maximum(jnp.sqrt(d2), 1e-12)
        C = out_ref.shape[1]
        out_ref[...] = (dots[:, :C] - aux[:, :C]) * scale


def kernel(support_feats, support_labels, query_feats):
    E = support_feats.shape[-1]
    C = support_labels.shape[-1]
    feats = support_feats.reshape(-1, E)
    labels = support_labels.reshape(-1, C).T
    q = query_feats.reshape(-1, E)
    nq = q.shape[0]
    n = feats.shape[0]

    B = 4096
    nblk = nq // B
    out = pl.pallas_call(
        _fused_body,
        grid=(1 + nblk,),
        in_specs=[
            pl.BlockSpec((n, E), lambda i: (0, 0)),
            pl.BlockSpec((C, n), lambda i: (0, 0)),
            pl.BlockSpec((B, E), lambda i: (jnp.maximum(i - 1, 0), 0)),
        ],
        out_specs=pl.BlockSpec((B, C), lambda i: (jnp.maximum(i - 1, 0), 0)),
        out_shape=jax.ShapeDtypeStruct((nq, C), jnp.float32),
        scratch_shapes=[pltpu.VMEM((_PAD, E), jnp.float32)],
        compiler_params=pltpu.CompilerParams(
            vmem_limit_bytes=63 * 1024 * 1024),
    )(feats, labels, q)
    return out
